# baseline (speedup 1.0000x reference)
import jax
import jax.numpy as jnp
from jax import lax
from jax.experimental import pallas as pl
from jax.experimental.pallas import tpu as pltpu
from jax.experimental.pallas import tpu_sc as plsc

_ROWS = 4096 * 50
_NC, _NS = 2, 16
_NW = _NC * _NS
_RPW = _ROWS // _NW
_R = 400
_NCHUNK = _RPW // _R

_mesh = plsc.VectorSubcoreMesh(core_axis_name="c", subcore_axis_name="s")

_out_t = jax.ShapeDtypeStruct((_ROWS, 32), jnp.float32)


def _body(x_hbm, m_hbm, t_hbm, v_hbm, s_hbm, b0, b1, rs0, rs1, ws0, ws1):
    outs = (m_hbm, t_hbm, v_hbm, s_hbm)
    bufs = (b0, b1)
    rsems = (rs0, rs1)
    wsems = (ws0, ws1)
    wid = lax.axis_index("s") * _NC + lax.axis_index("c")
    base = wid * _RPW

    def start_read(c, b):
        pltpu.async_copy(x_hbm.at[pl.ds(base + c * _R, _R)], bufs[b], rsems[b])

    def wait_read(b):
        pltpu.make_async_copy(
            x_hbm.at[pl.ds(0, _R)], bufs[b], rsems[b]
        ).wait()

    def start_writes(c, b):
        for k in range(4):
            pltpu.async_copy(
                bufs[b].at[:, pl.ds(32 * k, 32)],
                outs[k].at[pl.ds(base + c * _R, _R)],
                wsems[b],
            )

    def wait_writes(b):
        for k in range(4):
            pltpu.make_async_copy(
                bufs[b].at[:, pl.ds(32 * k, 32)],
                outs[k].at[pl.ds(0, _R)],
                wsems[b],
            ).wait()

    start_read(0, 0)
    for c in range(_NCHUNK):
        b = c % 2
        nb = 1 - b
        if c + 1 < _NCHUNK:
            if c >= 1:
                wait_writes(nb)
            start_read(c + 1, nb)
        wait_read(b)
        start_writes(c, b)
    wait_writes(0)
    wait_writes(1)


_split = pl.kernel(
    _body,
    out_type=(_out_t,) * 4,
    mesh=_mesh,
    scratch_types=[pltpu.VMEM((_R, 128), jnp.float32) for _ in range(2)]
    + [pltpu.SemaphoreType.DMA for _ in range(4)],
    compiler_params=pltpu.CompilerParams(use_tc_tiling_on_sc=False),
)


@jax.jit
def kernel(x):
    xr = x.reshape(_ROWS, 128)
    m, t, v, s = _split(xr)
    shp = (4096, 50, 32)
    return (m.reshape(shp), t.reshape(shp), v.reshape(shp), s.reshape(shp))
